# Initial kernel scaffold; baseline (speedup 1.0000x reference)
import jax
import jax.numpy as jnp
from jax.experimental import pallas as pl


def kernel(image):
    raise NotImplementedError("write your pallas kernel here")


# speedup vs baseline: 54.5001x; 54.5001x over previous
import functools

import numpy as np
import jax
import jax.numpy as jnp
from jax.experimental import pallas as pl
from jax.experimental.pallas import tpu as pltpu

H = W = 224
N = H * W
_RADII = [1, 2, 3, 4, 5, 6]
_RW = [1.0, 0.6, 0.4, 0.3, 0.2, 0.1]
_TAU = 0.15


def _build_static():
    d = {}
    for r, w in zip(_RADII, _RW):
        for dy in range(-r, r + 1):
            for dx in range(-r, r + 1):
                if (dx == 0 and dy == 0) or dx * dx + dy * dy > r * r:
                    continue
                d[(dy, dx)] = d.get((dy, dx), 0.0) + w
    offs = sorted(d.items(), key=lambda kv: kv[0][0] * W + kv[0][1])
    slots = []
    diag_j = None
    for (dy, dx), w in offs:
        if dy * W + dx > 0 and diag_j is None:
            diag_j = len(slots)
            slots.append((0, 0, None))
        slots.append((dy, dx, w))
    if diag_j is None:
        diag_j = len(slots)
        slots.append((0, 0, None))
    k = len(slots)

    dys = np.array([s[0] for s in slots], np.int64)
    dxs = np.array([s[1] for s in slots], np.int64)
    deltas = dys * W + dxs

    yy, xx = np.meshgrid(np.arange(H, dtype=np.int64),
                         np.arange(W, dtype=np.int64), indexing="ij")
    yf = yy.reshape(-1)
    xf = xx.reshape(-1)
    ny = yf[:, None] + dys[None, :]
    nx = xf[:, None] + dxs[None, :]
    valid = (ny >= 0) & (ny < H) & (nx >= 0) & (nx < W)

    pix = np.arange(N, dtype=np.int64)
    rows = np.broadcast_to(pix[:, None], (N, k))[valid]
    cols = (pix[:, None] + deltas[None, :])[valid]
    jj = np.broadcast_to(np.arange(k, dtype=np.int64)[None, :], (N, k))
    d2idx = (yf[:, None] * k + jj) * W + xf[:, None]
    sel = d2idx[valid]

    indices = np.stack([rows, cols]).astype(np.int32)
    return slots, diag_j, k, indices, sel.astype(np.int32)


_SLOTS, _DIAG_J, _K, _INDICES, _SEL = _build_static()
_NNZ = _SEL.shape[0]

_BY = 8
_WPAD = 256


def _stencil_body(planes_ref, out_ref):
    yb = pl.program_id(0) * _BY
    tall = [planes_ref[ch, pl.ds(yb, _BY + 16), :] for ch in range(3)]
    ctr = [t[6:6 + _BY, 6:6 + W] for t in tall]
    rowi = jax.lax.broadcasted_iota(jnp.int32, (_BY, W), 0) + yb
    coli = jax.lax.broadcasted_iota(jnp.int32, (_BY, W), 1)
    acc = jnp.zeros((_BY, W), jnp.float32)
    for j, (dy, dx, w) in enumerate(_SLOTS):
        if w is None:
            continue
        sh = [t[6 + dy:6 + dy + _BY, 6 + dx:6 + dx + W] for t in tall]
        d2 = ((ctr[0] - sh[0]) ** 2 + (ctr[1] - sh[1]) ** 2
              + (ctr[2] - sh[2]) ** 2)
        ew = w * jnp.exp(jnp.sqrt(d2) * (-1.0 / _TAU))
        ok = (coli >= -dx) & (coli < W - dx)
        if dy > 0:
            ok &= rowi < H - dy
        elif dy < 0:
            ok &= rowi >= -dy
        ewm = jnp.where(ok, ew, 0.0)
        acc = acc + ewm
        out_ref[:, j, :] = -ewm
    out_ref[:, _DIAG_J, :] = acc


def _dense_weights(image, interpret=False):
    img = image.astype(jnp.float32)
    planes = jnp.transpose(img, (2, 0, 1))
    planes = jnp.pad(planes, ((0, 0), (6, _BY + 10), (6, _WPAD - W - 6)))
    return pl.pallas_call(
        _stencil_body,
        grid=(H // _BY,),
        in_specs=[pl.BlockSpec((3, H + _BY + 16, _WPAD), lambda i: (0, 0, 0))],
        out_specs=pl.BlockSpec((_BY, _K, W), lambda i: (i, 0, 0)),
        out_shape=jax.ShapeDtypeStruct((H, _K, W), jnp.float32),
        interpret=interpret,
    )(planes)


def kernel(image):
    d2 = _dense_weights(image)
    vals = jnp.take(d2.reshape(-1), jnp.asarray(_SEL))
    return jnp.asarray(_INDICES), vals
